# baseline (speedup 1.0000x reference)
import jax
import jax.numpy as jnp
from jax.experimental import pallas as pl
from jax.experimental.pallas import tpu as pltpu

N = 4096
D = 128
TILE_R = 1024
TILE_C = 4096


def _seg_kernel(sr_ref, sc_ref, zi_ref, zj_ref, kr_ref, kc_ref, out_ref):
    i = pl.program_id(0)
    j = pl.program_id(1)

    r_lo = sr_ref[0, i]
    r_hi = sr_ref[1, i]
    c_lo = sc_ref[0, j]
    c_hi = sc_ref[1, j]
    overlap = (r_hi >= c_lo) & (c_hi >= r_lo)

    @pl.when(overlap)
    def _compute():
        gram = jax.lax.dot_general(
            zi_ref[...], zj_ref[...],
            dimension_numbers=(((1,), (1,)), ((), ())),
            preferred_element_type=jnp.float32,
        )
        mask = kr_ref[...] == kc_ref[...]
        out_ref[...] = jnp.where(mask, gram, 0.0)

    @pl.when(~overlap)
    def _zero():
        out_ref[...] = jnp.zeros((TILE_R, TILE_C), jnp.float32)

    crosses_diag = (i * TILE_R < (j + 1) * TILE_C) & (j * TILE_C < (i + 1) * TILE_R)

    @pl.when(crosses_diag)
    def _zero_diag():
        r = jax.lax.broadcasted_iota(jnp.int32, (TILE_R, TILE_C), 0) + i * TILE_R
        c = jax.lax.broadcasted_iota(jnp.int32, (TILE_R, TILE_C), 1) + j * TILE_C
        out_ref[...] = jnp.where(r == c, 0.0, out_ref[...])


def kernel(z, cls_label, batch):
    valid = ~((cls_label == 24) | (cls_label == 25) | (cls_label == 26))
    key = batch * 64 + cls_label
    key_row = jnp.where(valid, key, -1).reshape(N, 1)
    key_col = jnp.where(valid, key, -2).reshape(1, N)
    ep_row = jnp.stack([batch[::TILE_R], batch[TILE_R - 1::TILE_R]])
    ep_col = jnp.stack([batch[::TILE_C], batch[TILE_C - 1::TILE_C]])
    grid = (N // TILE_R, N // TILE_C)
    grid_spec = pltpu.PrefetchScalarGridSpec(
        num_scalar_prefetch=2,
        grid=grid,
        in_specs=[
            pl.BlockSpec((TILE_R, D), lambda i, j, sr, sc: (i, 0)),
            pl.BlockSpec((TILE_C, D), lambda i, j, sr, sc: (j, 0)),
            pl.BlockSpec((TILE_R, 1), lambda i, j, sr, sc: (i, 0)),
            pl.BlockSpec((1, TILE_C), lambda i, j, sr, sc: (0, j)),
        ],
        out_specs=pl.BlockSpec((TILE_R, TILE_C), lambda i, j, sr, sc: (i, j)),
    )
    return pl.pallas_call(
        _seg_kernel,
        grid_spec=grid_spec,
        out_shape=jax.ShapeDtypeStruct((N, N), jnp.float32),
    )(ep_row, ep_col, z, z, key_row, key_col)


# speedup vs baseline: 1.0121x; 1.0121x over previous
import jax
import jax.numpy as jnp
from jax.experimental import pallas as pl
from jax.experimental.pallas import tpu as pltpu

N = 4096
D = 128
TILE_R = 2048
TILE_C = 2048


def _seg_kernel(sr_ref, sc_ref, zi_ref, zj_ref, kr_ref, kc_ref, out_ref):
    i = pl.program_id(0)
    j = pl.program_id(1)

    r_lo = sr_ref[0, i]
    r_hi = sr_ref[1, i]
    c_lo = sc_ref[0, j]
    c_hi = sc_ref[1, j]
    overlap = (r_hi >= c_lo) & (c_hi >= r_lo)

    @pl.when(overlap)
    def _compute():
        gram = jax.lax.dot_general(
            zi_ref[...], zj_ref[...],
            dimension_numbers=(((1,), (1,)), ((), ())),
            preferred_element_type=jnp.float32,
        )
        mask = kr_ref[...] == kc_ref[...]
        out_ref[...] = jnp.where(mask, gram, 0.0)

    @pl.when(~overlap)
    def _zero():
        out_ref[...] = jnp.zeros((TILE_R, TILE_C), jnp.float32)

    crosses_diag = (i * TILE_R < (j + 1) * TILE_C) & (j * TILE_C < (i + 1) * TILE_R)

    @pl.when(crosses_diag)
    def _zero_diag():
        r = jax.lax.broadcasted_iota(jnp.int32, (TILE_R, TILE_C), 0) + i * TILE_R
        c = jax.lax.broadcasted_iota(jnp.int32, (TILE_R, TILE_C), 1) + j * TILE_C
        out_ref[...] = jnp.where(r == c, 0.0, out_ref[...])


def kernel(z, cls_label, batch):
    valid = ~((cls_label == 24) | (cls_label == 25) | (cls_label == 26))
    key = batch * 64 + cls_label
    key_row = jnp.where(valid, key, -1).reshape(N, 1)
    key_col = jnp.where(valid, key, -2).reshape(1, N)
    ep_row = jnp.stack([batch[::TILE_R], batch[TILE_R - 1::TILE_R]])
    ep_col = jnp.stack([batch[::TILE_C], batch[TILE_C - 1::TILE_C]])
    grid = (N // TILE_R, N // TILE_C)
    grid_spec = pltpu.PrefetchScalarGridSpec(
        num_scalar_prefetch=2,
        grid=grid,
        in_specs=[
            pl.BlockSpec((TILE_R, D), lambda i, j, sr, sc: (i, 0)),
            pl.BlockSpec((TILE_C, D), lambda i, j, sr, sc: (j, 0)),
            pl.BlockSpec((TILE_R, 1), lambda i, j, sr, sc: (i, 0)),
            pl.BlockSpec((1, TILE_C), lambda i, j, sr, sc: (0, j)),
        ],
        out_specs=pl.BlockSpec((TILE_R, TILE_C), lambda i, j, sr, sc: (i, j)),
    )
    zh = z.astype(jnp.bfloat16)
    return pl.pallas_call(
        _seg_kernel,
        grid_spec=grid_spec,
        out_shape=jax.ShapeDtypeStruct((N, N), jnp.float32),
    )(ep_row, ep_col, zh, zh, key_row, key_col)


# speedup vs baseline: 1.1001x; 1.0870x over previous
import jax
import jax.numpy as jnp
from jax.experimental import pallas as pl
from jax.experimental.pallas import tpu as pltpu

N = 4096
D = 128
TILE_R = 2048
TILE_C = 2048


def _seg_kernel(sr_ref, sc_ref, zi_ref, zj_ref, kr_ref, kc_ref, out_ref):
    i = pl.program_id(0)
    j = pl.program_id(1)

    r_lo = sr_ref[0, i]
    r_hi = sr_ref[1, i]
    c_lo = sc_ref[0, j]
    c_hi = sc_ref[1, j]
    overlap = (r_hi >= c_lo) & (c_hi >= r_lo)

    @pl.when(overlap)
    def _compute():
        gram = jax.lax.dot_general(
            zi_ref[...], zj_ref[...],
            dimension_numbers=(((1,), (1,)), ((), ())),
            preferred_element_type=jnp.float32,
        )
        mask = kr_ref[...] == kc_ref[...]
        out_ref[...] = jnp.where(mask, gram, 0.0)

    @pl.when(~overlap)
    def _zero():
        out_ref[...] = jnp.zeros((TILE_R, TILE_C), jnp.float32)

    off = i * TILE_R - j * TILE_C

    @pl.when((off > -TILE_C) & (off < TILE_R))
    def _zero_diag():
        eye = (jax.lax.broadcasted_iota(jnp.int32, (128, 128), 0)
               == jax.lax.broadcasted_iota(jnp.int32, (128, 128), 1))
        for k in range(TILE_R // 128):
            c0 = k * 128 + off

            @pl.when((c0 >= 0) & (c0 < TILE_C))
            def _blk():
                rs = pl.ds(k * 128, 128)
                cs = pl.ds(c0, 128)
                out_ref[rs, cs] = jnp.where(eye, 0.0, out_ref[rs, cs])


def kernel(z, cls_label, batch):
    valid = ~((cls_label == 24) | (cls_label == 25) | (cls_label == 26))
    key = batch * 64 + cls_label
    key_row = jnp.where(valid, key, -1).reshape(N, 1)
    key_col = jnp.where(valid, key, -2).reshape(1, N)
    ep_row = jnp.stack([batch[::TILE_R], batch[TILE_R - 1::TILE_R]])
    ep_col = jnp.stack([batch[::TILE_C], batch[TILE_C - 1::TILE_C]])
    grid = (N // TILE_R, N // TILE_C)
    grid_spec = pltpu.PrefetchScalarGridSpec(
        num_scalar_prefetch=2,
        grid=grid,
        in_specs=[
            pl.BlockSpec((TILE_R, D), lambda i, j, sr, sc: (i, 0)),
            pl.BlockSpec((TILE_C, D), lambda i, j, sr, sc: (j, 0)),
            pl.BlockSpec((TILE_R, 1), lambda i, j, sr, sc: (i, 0)),
            pl.BlockSpec((1, TILE_C), lambda i, j, sr, sc: (0, j)),
        ],
        out_specs=pl.BlockSpec((TILE_R, TILE_C), lambda i, j, sr, sc: (i, j)),
    )
    return pl.pallas_call(
        _seg_kernel,
        grid_spec=grid_spec,
        out_shape=jax.ShapeDtypeStruct((N, N), jnp.float32),
    )(ep_row, ep_col, z, z, key_row, key_col)
